# baseline (speedup 1.0000x reference)
import functools

import jax
from jax import lax
import jax.numpy as jnp
from jax.experimental import pallas as pl
from jax.experimental.pallas import tpu as pltpu
from jax.experimental.pallas import tpu_sc as plsc

_C = 64
_D = 64
_TEMP = 10.0

_NC = 2
_NS = 16
_NW = _NC * _NS
_CH = 128

_MM = (((1,), (0,)), ((), ()))
_MM_T = (((0,), (0,)), ((), ()))


def _row_form(v):
  r = jax.lax.broadcasted_iota(jnp.int32, (_C, _C), 0)
  c = jax.lax.broadcasted_iota(jnp.int32, (_C, _C), 1)
  eye = (r == c).astype(jnp.float32)
  return jnp.sum(eye * v, axis=0, keepdims=True)


def _seg_sc(feat1, lab1, feat2, lab2, rows_per_w):
  nstage = 4
  stage_rows = rows_per_w // nstage
  mesh = plsc.VectorSubcoreMesh(core_axis_name="c", subcore_axis_name="s")
  out_type = (
      jax.ShapeDtypeStruct((_NW, _C * _D), jnp.float32),
      jax.ShapeDtypeStruct((_NW, _C * 16), jnp.float32),
      jax.ShapeDtypeStruct((_NW, _C * _D), jnp.float32),
      jax.ShapeDtypeStruct((_NW, _C * 16), jnp.float32),
  )

  @functools.partial(
      pl.kernel, mesh=mesh, out_type=out_type,
      compiler_params=pltpu.CompilerParams(needs_layout_passes=False),
      scratch_types=[
          pltpu.VMEM((stage_rows, _D), jnp.float32),
          pltpu.VMEM((rows_per_w,), jnp.int32),
          pltpu.VMEM((_C * _D,), jnp.float32),
          pltpu.VMEM((_C * 16,), jnp.float32),
      ])
  def k(f1_hbm, l1_hbm, f2_hbm, l2_hbm,
        o_s1, o_c1, o_s2, o_c2, fbuf, labbuf, acc, accc):
    cid = lax.axis_index("c")
    sid = lax.axis_index("s")
    wid = cid * _NS + sid
    base = wid * rows_per_w

    zeros16 = jnp.zeros((16,), jnp.float32)
    ones16 = jnp.ones((16,), jnp.float32)
    iota16 = lax.iota(jnp.int32, 16)

    def run(f_hbm, l_hbm, o_s, o_c):
      for i in range(_C * _D // 16):
        acc[pl.ds(16 * i, 16)] = zeros16
      for i in range(_C):
        accc[pl.ds(16 * i, 16)] = zeros16
      pltpu.sync_copy(l_hbm.at[pl.ds(base, rows_per_w)], labbuf)
      for s in range(nstage):
        pltpu.sync_copy(
            f_hbm.at[pl.ds(base + s * stage_rows, stage_rows)], fbuf)

        def body(i, carry):
          labs = plsc.load_gather(
              labbuf, [jnp.zeros((16,), jnp.int32) + (s * stage_rows + i)])
          for t in range(_D // 16):
            v = plsc.load_gather(fbuf, [jnp.zeros((16,), jnp.int32) + i,
                                        16 * t + iota16])
            plsc.addupdate_scatter(acc, [labs * _D + 16 * t + iota16], v)
          plsc.addupdate_scatter(accc, [labs * 16 + iota16], ones16)
          return carry

        lax.fori_loop(0, stage_rows, body, 0)
      pltpu.sync_copy(acc, o_s.at[wid])
      pltpu.sync_copy(accc, o_c.at[wid])

    run(f1_hbm, l1_hbm, o_s1, o_c1)
    run(f2_hbm, l2_hbm, o_s2, o_c2)

  return k(feat1, lab1, feat2, lab2)


def _dist_kernel(lab1_ref, lab2_ref, f1_ref, f2_ref,
                 s1_ref, c1p_ref, s2_ref, c2p_ref, out_ref,
                 c1_ref, c2_ref, cnt1_ref, cnt2_ref, dsum1_ref, dsum2_ref,
                 *, nblocks, block):
  b = pl.program_id(0)

  lab1 = lab1_ref[0]
  lab2 = lab2_ref[0]
  f1 = f1_ref[...]
  f2 = f2_ref[...]

  iota = jax.lax.broadcasted_iota(jnp.int32, (_C, block), 0)
  oh1 = (lab1 == iota).astype(jnp.float32)
  oh2 = (lab2 == iota).astype(jnp.float32)

  @pl.when(b == 0)
  def _centers():
    cnt1 = jnp.sum(c1p_ref[...], axis=0)[:, 0:1]
    cnt2 = jnp.sum(c2p_ref[...], axis=0)[:, 0:1]
    cnt1_ref[...] = cnt1
    cnt2_ref[...] = cnt2
    safe1 = jnp.maximum(cnt1, 1.0)
    safe2 = jnp.maximum(cnt2, 1.0)
    c1_ref[...] = jnp.sum(s1_ref[...], axis=0) / safe1
    c2_ref[...] = jnp.sum(s2_ref[...], axis=0) / safe2
    dsum1_ref[...] = jnp.zeros_like(dsum1_ref)
    dsum2_ref[...] = jnp.zeros_like(dsum2_ref)

  cb1 = jax.lax.dot_general(
      oh1, c1_ref[...], _MM_T, preferred_element_type=jnp.float32)
  cb2 = jax.lax.dot_general(
      oh2, c2_ref[...], _MM_T, preferred_element_type=jnp.float32)
  diff1 = f1 - cb1
  diff2 = f2 - cb2
  ones_dd = jnp.ones((_D, _D), jnp.float32)
  dsqb1 = jax.lax.dot_general(
      diff1 * diff1, ones_dd, _MM, preferred_element_type=jnp.float32)
  dsqb2 = jax.lax.dot_general(
      diff2 * diff2, ones_dd, _MM, preferred_element_type=jnp.float32)
  d1 = jnp.sqrt(jnp.maximum(dsqb1, 1e-24))
  d2 = jnp.sqrt(jnp.maximum(dsqb2, 1e-24))
  dsum1_ref[...] += jax.lax.dot_general(
      oh1, d1, _MM, preferred_element_type=jnp.float32)
  dsum2_ref[...] += jax.lax.dot_general(
      oh2, d2, _MM, preferred_element_type=jnp.float32)

  @pl.when(b == nblocks - 1)
  def _final():
    cnt1 = cnt1_ref[...]
    cnt2 = cnt2_ref[...]
    safe1 = jnp.maximum(cnt1, 1.0)
    safe2 = jnp.maximum(cnt2, 1.0)
    mean_d1 = dsum1_ref[:, 0:1] / safe1
    mean_d2 = dsum2_ref[:, 0:1] / safe2
    mask_intra = jnp.logical_and(cnt1 > 1.0, cnt2 > 1.0)
    intra = jnp.sum(jnp.where(mask_intra, mean_d1 + mean_d2, 0.0),
                    keepdims=True)

    c1 = c1_ref[...]
    c2 = c2_ref[...]
    n1 = jnp.sum(c1 * c1, axis=1, keepdims=True)
    n2 = jnp.sum(c2 * c2, axis=1, keepdims=True)
    gram = jax.lax.dot_general(
        c1, c2, (((1,), (1,)), ((), ())),
        preferred_element_type=jnp.float32)
    dsq = n1 + _row_form(n2) - 2.0 * gram
    dmat = jnp.sqrt(jnp.maximum(dsq, 1e-24))

    mask = jnp.logical_and(cnt1 > 0.0, cnt2 > 0.0).astype(jnp.float32)
    n_valid = jnp.sum(mask, keepdims=True)
    rowsum = jnp.sum(dmat * mask, axis=0, keepdims=True)
    masked_total = jax.lax.dot_general(
        rowsum, mask, (((1,), (0,)), ((), ())),
        preferred_element_type=jnp.float32)
    pair_cnt = jnp.maximum(n_valid * n_valid, 1.0)
    inter = jnp.where(n_valid > 1.0, masked_total / pair_cnt, 0.0)

    normalized = intra / (inter + 1e-8)
    loss = jnp.where(inter > 0.0,
                     jnp.log(1.0 + jnp.exp(normalized / _TEMP)),
                     intra)
    out_ref[...] = loss


@functools.partial(jax.jit, static_argnames=("block",))
def _dist_loss(feat1, label1, feat2, label2, block=8192):
  n, d = feat1.shape
  nblocks = n // block
  rows_per_w = n // _NW
  l1 = label1.astype(jnp.int32)
  l2 = label2.astype(jnp.int32)

  s1p, c1p, s2p, c2p = _seg_sc(feat1, l1, feat2, l2, rows_per_w)
  s1p = s1p.reshape(_NW, _C, _D)
  s2p = s2p.reshape(_NW, _C, _D)
  c1p = c1p.reshape(_NW, _C, 16)
  c2p = c2p.reshape(_NW, _C, 16)

  lab1 = l1.reshape(nblocks, 1, block)
  lab2 = l2.reshape(nblocks, 1, block)
  out = pl.pallas_call(
      functools.partial(_dist_kernel, nblocks=nblocks, block=block),
      grid=(nblocks,),
      in_specs=[
          pl.BlockSpec((1, 1, block), lambda b: (b, 0, 0)),
          pl.BlockSpec((1, 1, block), lambda b: (b, 0, 0)),
          pl.BlockSpec((block, d), lambda b: (b, 0)),
          pl.BlockSpec((block, d), lambda b: (b, 0)),
          pl.BlockSpec((_NW, _C, _D), lambda b: (0, 0, 0)),
          pl.BlockSpec((_NW, _C, 16), lambda b: (0, 0, 0)),
          pl.BlockSpec((_NW, _C, _D), lambda b: (0, 0, 0)),
          pl.BlockSpec((_NW, _C, 16), lambda b: (0, 0, 0)),
      ],
      out_specs=pl.BlockSpec((1, 1), lambda b: (0, 0)),
      out_shape=jax.ShapeDtypeStruct((1, 1), jnp.float32),
      scratch_shapes=[
          pltpu.VMEM((_C, _D), jnp.float32),
          pltpu.VMEM((_C, _D), jnp.float32),
          pltpu.VMEM((_C, 1), jnp.float32),
          pltpu.VMEM((_C, 1), jnp.float32),
          pltpu.VMEM((_C, _D), jnp.float32),
          pltpu.VMEM((_C, _D), jnp.float32),
      ],
  )(lab1, lab2, feat1, feat2, s1p, c1p, s2p, c2p)
  return out[0, 0]


def kernel(feat1, label1, feat2, label2):
  return _dist_loss(feat1, label1, feat2, label2)


# speedup vs baseline: 1.0164x; 1.0164x over previous
import functools

import jax
from jax import lax
import jax.numpy as jnp
from jax.experimental import pallas as pl
from jax.experimental.pallas import tpu as pltpu
from jax.experimental.pallas import tpu_sc as plsc

_C = 64
_D = 64
_TEMP = 10.0

_NC = 2
_NS = 16
_NW = _NC * _NS
_CH = 128

_MM = (((1,), (0,)), ((), ()))
_MM_T = (((0,), (0,)), ((), ()))


def _row_form(v):
  r = jax.lax.broadcasted_iota(jnp.int32, (_C, _C), 0)
  c = jax.lax.broadcasted_iota(jnp.int32, (_C, _C), 1)
  eye = (r == c).astype(jnp.float32)
  return jnp.sum(eye * v, axis=0, keepdims=True)


def _seg_sc(feat1, lab1, feat2, lab2, rows_per_w):
  nstage = 4
  stage_rows = rows_per_w // nstage
  mesh = plsc.VectorSubcoreMesh(core_axis_name="c", subcore_axis_name="s")
  out_type = (
      jax.ShapeDtypeStruct((_NW, _C * _D), jnp.float32),
      jax.ShapeDtypeStruct((_NW, _C * 16), jnp.float32),
      jax.ShapeDtypeStruct((_NW, _C * _D), jnp.float32),
      jax.ShapeDtypeStruct((_NW, _C * 16), jnp.float32),
  )

  @functools.partial(
      pl.kernel, mesh=mesh, out_type=out_type,
      compiler_params=pltpu.CompilerParams(needs_layout_passes=False),
      scratch_types=[
          pltpu.VMEM((stage_rows, _D), jnp.float32),
          pltpu.VMEM((rows_per_w,), jnp.int32),
          pltpu.VMEM((_C * _D,), jnp.float32),
          pltpu.VMEM((_C * 16,), jnp.float32),
      ])
  def k(f1_hbm, l1_hbm, f2_hbm, l2_hbm,
        o_s1, o_c1, o_s2, o_c2, fbuf, labbuf, acc, accc):
    cid = lax.axis_index("c")
    sid = lax.axis_index("s")
    wid = cid * _NS + sid
    base = wid * rows_per_w

    zeros16 = jnp.zeros((16,), jnp.float32)
    ones16 = jnp.ones((16,), jnp.float32)
    iota16 = lax.iota(jnp.int32, 16)

    def run(f_hbm, l_hbm, o_s, o_c):
      for i in range(_C * _D // 16):
        acc[pl.ds(16 * i, 16)] = zeros16
      for i in range(_C):
        accc[pl.ds(16 * i, 16)] = zeros16
      pltpu.sync_copy(l_hbm.at[pl.ds(base, rows_per_w)], labbuf)
      for s in range(nstage):
        pltpu.sync_copy(
            f_hbm.at[pl.ds(base + s * stage_rows, stage_rows)], fbuf)

        def body(i, carry):
          for u in range(4):
            r = 4 * i + u
            labs = plsc.load_gather(
                labbuf, [jnp.zeros((16,), jnp.int32) + (s * stage_rows + r)])
            for t in range(_D // 16):
              v = plsc.load_gather(fbuf, [jnp.zeros((16,), jnp.int32) + r,
                                          16 * t + iota16])
              plsc.addupdate_scatter(acc, [labs * _D + 16 * t + iota16], v)
            plsc.addupdate_scatter(accc, [labs * 16 + iota16], ones16)
          return carry

        lax.fori_loop(0, stage_rows // 4, body, 0)
      pltpu.sync_copy(acc, o_s.at[wid])
      pltpu.sync_copy(accc, o_c.at[wid])

    run(f1_hbm, l1_hbm, o_s1, o_c1)
    run(f2_hbm, l2_hbm, o_s2, o_c2)

  return k(feat1, lab1, feat2, lab2)


def _dist_kernel(lab1_ref, lab2_ref, f1_ref, f2_ref,
                 s1_ref, c1p_ref, s2_ref, c2p_ref, out_ref,
                 c1_ref, c2_ref, cnt1_ref, cnt2_ref, dsum1_ref, dsum2_ref,
                 *, nblocks, block):
  b = pl.program_id(0)

  lab1 = lab1_ref[0]
  lab2 = lab2_ref[0]
  f1 = f1_ref[...]
  f2 = f2_ref[...]

  iota = jax.lax.broadcasted_iota(jnp.int32, (_C, block), 0)
  oh1 = (lab1 == iota).astype(jnp.float32)
  oh2 = (lab2 == iota).astype(jnp.float32)

  @pl.when(b == 0)
  def _centers():
    cnt1 = jnp.sum(c1p_ref[...], axis=0)[:, 0:1]
    cnt2 = jnp.sum(c2p_ref[...], axis=0)[:, 0:1]
    cnt1_ref[...] = cnt1
    cnt2_ref[...] = cnt2
    safe1 = jnp.maximum(cnt1, 1.0)
    safe2 = jnp.maximum(cnt2, 1.0)
    c1_ref[...] = jnp.sum(s1_ref[...], axis=0) / safe1
    c2_ref[...] = jnp.sum(s2_ref[...], axis=0) / safe2
    dsum1_ref[...] = jnp.zeros_like(dsum1_ref)
    dsum2_ref[...] = jnp.zeros_like(dsum2_ref)

  cb1 = jax.lax.dot_general(
      oh1, c1_ref[...], _MM_T, preferred_element_type=jnp.float32)
  cb2 = jax.lax.dot_general(
      oh2, c2_ref[...], _MM_T, preferred_element_type=jnp.float32)
  diff1 = f1 - cb1
  diff2 = f2 - cb2
  ones_dd = jnp.ones((_D, _D), jnp.float32)
  dsqb1 = jax.lax.dot_general(
      diff1 * diff1, ones_dd, _MM, preferred_element_type=jnp.float32)
  dsqb2 = jax.lax.dot_general(
      diff2 * diff2, ones_dd, _MM, preferred_element_type=jnp.float32)
  d1 = jnp.sqrt(jnp.maximum(dsqb1, 1e-24))
  d2 = jnp.sqrt(jnp.maximum(dsqb2, 1e-24))
  dsum1_ref[...] += jax.lax.dot_general(
      oh1, d1, _MM, preferred_element_type=jnp.float32)
  dsum2_ref[...] += jax.lax.dot_general(
      oh2, d2, _MM, preferred_element_type=jnp.float32)

  @pl.when(b == nblocks - 1)
  def _final():
    cnt1 = cnt1_ref[...]
    cnt2 = cnt2_ref[...]
    safe1 = jnp.maximum(cnt1, 1.0)
    safe2 = jnp.maximum(cnt2, 1.0)
    mean_d1 = dsum1_ref[:, 0:1] / safe1
    mean_d2 = dsum2_ref[:, 0:1] / safe2
    mask_intra = jnp.logical_and(cnt1 > 1.0, cnt2 > 1.0)
    intra = jnp.sum(jnp.where(mask_intra, mean_d1 + mean_d2, 0.0),
                    keepdims=True)

    c1 = c1_ref[...]
    c2 = c2_ref[...]
    n1 = jnp.sum(c1 * c1, axis=1, keepdims=True)
    n2 = jnp.sum(c2 * c2, axis=1, keepdims=True)
    gram = jax.lax.dot_general(
        c1, c2, (((1,), (1,)), ((), ())),
        preferred_element_type=jnp.float32)
    dsq = n1 + _row_form(n2) - 2.0 * gram
    dmat = jnp.sqrt(jnp.maximum(dsq, 1e-24))

    mask = jnp.logical_and(cnt1 > 0.0, cnt2 > 0.0).astype(jnp.float32)
    n_valid = jnp.sum(mask, keepdims=True)
    rowsum = jnp.sum(dmat * mask, axis=0, keepdims=True)
    masked_total = jax.lax.dot_general(
        rowsum, mask, (((1,), (0,)), ((), ())),
        preferred_element_type=jnp.float32)
    pair_cnt = jnp.maximum(n_valid * n_valid, 1.0)
    inter = jnp.where(n_valid > 1.0, masked_total / pair_cnt, 0.0)

    normalized = intra / (inter + 1e-8)
    loss = jnp.where(inter > 0.0,
                     jnp.log(1.0 + jnp.exp(normalized / _TEMP)),
                     intra)
    out_ref[...] = loss


@functools.partial(jax.jit, static_argnames=("block",))
def _dist_loss(feat1, label1, feat2, label2, block=8192):
  n, d = feat1.shape
  nblocks = n // block
  rows_per_w = n // _NW
  l1 = label1.astype(jnp.int32)
  l2 = label2.astype(jnp.int32)

  s1p, c1p, s2p, c2p = _seg_sc(feat1, l1, feat2, l2, rows_per_w)
  s1p = s1p.reshape(_NW, _C, _D)
  s2p = s2p.reshape(_NW, _C, _D)
  c1p = c1p.reshape(_NW, _C, 16)
  c2p = c2p.reshape(_NW, _C, 16)

  lab1 = l1.reshape(nblocks, 1, block)
  lab2 = l2.reshape(nblocks, 1, block)
  out = pl.pallas_call(
      functools.partial(_dist_kernel, nblocks=nblocks, block=block),
      grid=(nblocks,),
      in_specs=[
          pl.BlockSpec((1, 1, block), lambda b: (b, 0, 0)),
          pl.BlockSpec((1, 1, block), lambda b: (b, 0, 0)),
          pl.BlockSpec((block, d), lambda b: (b, 0)),
          pl.BlockSpec((block, d), lambda b: (b, 0)),
          pl.BlockSpec((_NW, _C, _D), lambda b: (0, 0, 0)),
          pl.BlockSpec((_NW, _C, 16), lambda b: (0, 0, 0)),
          pl.BlockSpec((_NW, _C, _D), lambda b: (0, 0, 0)),
          pl.BlockSpec((_NW, _C, 16), lambda b: (0, 0, 0)),
      ],
      out_specs=pl.BlockSpec((1, 1), lambda b: (0, 0)),
      out_shape=jax.ShapeDtypeStruct((1, 1), jnp.float32),
      scratch_shapes=[
          pltpu.VMEM((_C, _D), jnp.float32),
          pltpu.VMEM((_C, _D), jnp.float32),
          pltpu.VMEM((_C, 1), jnp.float32),
          pltpu.VMEM((_C, 1), jnp.float32),
          pltpu.VMEM((_C, _D), jnp.float32),
          pltpu.VMEM((_C, _D), jnp.float32),
      ],
  )(lab1, lab2, feat1, feat2, s1p, c1p, s2p, c2p)
  return out[0, 0]


def kernel(feat1, label1, feat2, label2):
  return _dist_loss(feat1, label1, feat2, label2)
